# baseline (speedup 1.0000x reference)
import functools

import jax
import jax.numpy as jnp
from jax.experimental import pallas as pl
from jax.experimental.pallas import tpu as pltpu

N_NODES = 10000
BLK = 512
MPAD = 10240


def _layer_body(agg_ref, x_ref, wl_ref, wr_ref, b_ref, o_ref):
    acc = jnp.dot(agg_ref[...], wl_ref[...], preferred_element_type=jnp.float32)
    acc = acc + jnp.dot(x_ref[...], wr_ref[...], preferred_element_type=jnp.float32)
    o_ref[...] = jnp.maximum(acc + b_ref[...], 0.0)


def _head_body(agg_ref, h_ref, wl_ref, wr_ref, b_ref, wlin_ref, blin_ref, o_ref):
    acc = jnp.dot(agg_ref[...], wl_ref[...], preferred_element_type=jnp.float32)
    acc = acc + jnp.dot(h_ref[...], wr_ref[...], preferred_element_type=jnp.float32)
    h2 = jnp.maximum(acc + b_ref[...], 0.0)
    o_ref[...] = jnp.dot(h2, wlin_ref[...], preferred_element_type=jnp.float32) + blin_ref[...]


def _row_spec(k):
    return pl.BlockSpec((BLK, k), lambda i: (i, 0))


def _full_spec(r, c):
    return pl.BlockSpec((r, c), lambda i: (0, 0))


def _sage_layer(agg, x, W_l, W_r, b):
    m, k = x.shape
    h = W_l.shape[1]
    return pl.pallas_call(
        _layer_body,
        grid=(m // BLK,),
        in_specs=[_row_spec(k), _row_spec(k), _full_spec(k, h), _full_spec(k, h),
                  _full_spec(1, h)],
        out_specs=_row_spec(h),
        out_shape=jax.ShapeDtypeStruct((m, h), jnp.float32),
    )(agg, x, W_l, W_r, b.reshape(1, h))


def _sage_head(agg, h1, W_l, W_r, b, W_lin, b_lin):
    m, k = h1.shape
    h = W_l.shape[1]
    d_out = W_lin.shape[1]
    return pl.pallas_call(
        _head_body,
        grid=(m // BLK,),
        in_specs=[_row_spec(k), _row_spec(k), _full_spec(k, h), _full_spec(k, h),
                  _full_spec(1, h), _full_spec(h, d_out), _full_spec(1, d_out)],
        out_specs=_row_spec(d_out),
        out_shape=jax.ShapeDtypeStruct((m, d_out), jnp.float32),
    )(agg, h1, W_l, W_r, b.reshape(1, h), W_lin, b_lin.reshape(1, d_out))


def _segment_mean(x, src, dst, inv_cnt):
    msg = jnp.take(x, src, axis=0)
    agg = jax.ops.segment_sum(msg, dst, num_segments=N_NODES)
    return agg * inv_cnt


def kernel(x, edge_index, W1_l, b1, W1_r, W2_l, b2, W2_r, W_lin, b_lin):
    src = edge_index[0].astype(jnp.int32)
    dst = edge_index[1].astype(jnp.int32)
    cnt = jax.ops.segment_sum(jnp.ones((src.shape[0],), jnp.float32), dst,
                              num_segments=N_NODES)
    inv_cnt = (1.0 / jnp.clip(cnt, 1.0, None))[:, None]

    agg1 = _segment_mean(x, src, dst, inv_cnt)
    xp = jnp.pad(x, ((0, MPAD - N_NODES), (0, 0)))
    agg1p = jnp.pad(agg1, ((0, MPAD - N_NODES), (0, 0)))
    h1 = _sage_layer(agg1p, xp, W1_l, W1_r, b1)[:N_NODES]

    agg2 = _segment_mean(h1, src, dst, inv_cnt)
    h1p = jnp.pad(h1, ((0, MPAD - N_NODES), (0, 0)))
    agg2p = jnp.pad(agg2, ((0, MPAD - N_NODES), (0, 0)))
    out = _sage_head(agg2p, h1p, W2_l, W2_r, b2, W_lin, b_lin)[:N_NODES]
    return out


# speedup vs baseline: 3.0670x; 3.0670x over previous
import functools

import jax
import jax.numpy as jnp
from jax import lax
from jax.experimental import pallas as pl
from jax.experimental.pallas import tpu as pltpu
from jax.experimental.pallas import tpu_sc as plsc

N = 10000
E = 160000
EP = 161792
ET = EP // 16
K = 128
NCH = ET // K
NCH0 = 40

DH = 128

ACC = N + 8
FL = 640
FCH = 40

BLK = 400


def _edge_scan(s, src_hbm, dst_hbm, table, acc, src_v, dst_v, rows_v, sem):
    def chunk(ci, carry):
        b = s * ET + ci * K
        pltpu.sync_copy(src_hbm.at[pl.ds(b, K)], src_v)
        pltpu.sync_copy(dst_hbm.at[pl.ds(b, K)], dst_v)
        pltpu.async_copy(table.at[plsc.Indices(src_v)], rows_v, sem).wait()
        pltpu.sync_copy(rows_v, acc.at[plsc.Indices(dst_v)], add=True)
        return carry

    lax.fori_loop(0, NCH, chunk, 0)


def _cnt_scan(s, dst_hbm, acc, dst_v, rows_v, ch_lo, ch_hi):
    def chunk(ci, carry):
        b = s * ET + ci * K
        pltpu.sync_copy(dst_hbm.at[pl.ds(b, K)], dst_v)
        pltpu.sync_copy(rows_v, acc.at[plsc.Indices(dst_v)], add=True)
        return carry

    lax.fori_loop(ch_lo, ch_hi, chunk, 0)


def _zero_acc(start, acc, stage_v):
    for i in range(FL // FCH):
        pltpu.sync_copy(stage_v.at[pl.ds(0, FCH)],
                        acc.at[pl.ds(start + i * FCH, FCH)])


def _flush_acc(start, acc, stage_v, out_hbm):
    for i in range(FL // FCH):
        pltpu.sync_copy(acc.at[pl.ds(start + i * FCH, FCH)],
                        stage_v.at[pl.ds(0, FCH)])
        pltpu.sync_copy(stage_v.at[pl.ds(0, FCH)],
                        out_hbm.at[pl.ds(start + i * FCH, FCH)])


def _agg1_call(x0, x1, srcp, dstp, zF, oF):
    mesh = plsc.VectorSubcoreMesh(core_axis_name="c", subcore_axis_name="s")

    @functools.partial(
        pl.kernel,
        out_type=[jax.ShapeDtypeStruct((N, DH), jnp.float32)] * 4,
        mesh=mesh,
        scratch_types=[
            pltpu.VMEM_SHARED((ACC, DH), jnp.float32),
            pltpu.VMEM((K,), jnp.int32),
            pltpu.VMEM((K,), jnp.int32),
            pltpu.VMEM((K, DH), jnp.float32),
            pltpu.VMEM((FCH, DH), jnp.float32),
            pltpu.SemaphoreType.DMA,
        ],
    )
    def k(x0_h, x1_h, src_h, dst_h, zf_h, of_h,
          agg0_o, agg1_o, cnta_o, cntb_o,
          acc, src_v, dst_v, rows_v, stage_v, sem):
        c = lax.axis_index("c")
        s = lax.axis_index("s")
        start = jnp.minimum(s * FL, N - FL)

        pltpu.sync_copy(zf_h, stage_v)
        _zero_acc(start, acc, stage_v)
        plsc.subcore_barrier()

        @pl.when(c == 0)
        def _():
            _edge_scan(s, src_h, dst_h, x0_h, acc, src_v, dst_v, rows_v, sem)

        @pl.when(c == 1)
        def _():
            _edge_scan(s, src_h, dst_h, x1_h, acc, src_v, dst_v, rows_v, sem)

        plsc.subcore_barrier()

        @pl.when(c == 0)
        def _():
            _flush_acc(start, acc, stage_v, agg0_o)

        @pl.when(c == 1)
        def _():
            _flush_acc(start, acc, stage_v, agg1_o)

        plsc.subcore_barrier()

        pltpu.sync_copy(zf_h, stage_v)
        _zero_acc(start, acc, stage_v)
        pltpu.sync_copy(of_h, rows_v)
        plsc.subcore_barrier()

        @pl.when(c == 0)
        def _():
            _cnt_scan(s, dst_h, acc, dst_v, rows_v, 0, NCH0)

        @pl.when(c == 1)
        def _():
            _cnt_scan(s, dst_h, acc, dst_v, rows_v, NCH0, NCH)

        plsc.subcore_barrier()

        @pl.when(c == 0)
        def _():
            _flush_acc(start, acc, stage_v, cnta_o)

        @pl.when(c == 1)
        def _():
            _flush_acc(start, acc, stage_v, cntb_o)

    return k(x0, x1, srcp, dstp, zF, oF)


def _agg2_call(q0, q1, q2, q3, srcp, dstp, zF):
    mesh = plsc.VectorSubcoreMesh(core_axis_name="c", subcore_axis_name="s")

    @functools.partial(
        pl.kernel,
        out_type=[jax.ShapeDtypeStruct((N, DH), jnp.float32)] * 4,
        mesh=mesh,
        scratch_types=[
            pltpu.VMEM_SHARED((ACC, DH), jnp.float32),
            pltpu.VMEM((K,), jnp.int32),
            pltpu.VMEM((K,), jnp.int32),
            pltpu.VMEM((K, DH), jnp.float32),
            pltpu.VMEM((FCH, DH), jnp.float32),
            pltpu.SemaphoreType.DMA,
        ],
    )
    def k(q0_h, q1_h, q2_h, q3_h, src_h, dst_h, zf_h,
          a0_o, a1_o, a2_o, a3_o,
          acc, src_v, dst_v, rows_v, stage_v, sem):
        c = lax.axis_index("c")
        s = lax.axis_index("s")
        start = jnp.minimum(s * FL, N - FL)
        for qa, qb, oa, ob in [(q0_h, q2_h, a0_o, a2_o),
                               (q1_h, q3_h, a1_o, a3_o)]:
            pltpu.sync_copy(zf_h, stage_v)
            _zero_acc(start, acc, stage_v)
            plsc.subcore_barrier()

            @pl.when(c == 0)
            def _():
                _edge_scan(s, src_h, dst_h, qa, acc, src_v, dst_v, rows_v, sem)

            @pl.when(c == 1)
            def _():
                _edge_scan(s, src_h, dst_h, qb, acc, src_v, dst_v, rows_v, sem)

            plsc.subcore_barrier()

            @pl.when(c == 0)
            def _():
                _flush_acc(start, acc, stage_v, oa)

            @pl.when(c == 1)
            def _():
                _flush_acc(start, acc, stage_v, ob)

            plsc.subcore_barrier()

    return k(q0, q1, q2, q3, srcp, dstp, zF)


def _layer1_body(a0_ref, a1_ref, ca_ref, cb_ref, x_ref, wl_ref, wr_ref, b_ref,
                 q0_ref, q1_ref, q2_ref, q3_ref):
    inv = 1.0 / jnp.maximum(ca_ref[:, 0:1] + cb_ref[:, 0:1], 1.0)
    wl = wl_ref[...]
    z = jnp.dot(a0_ref[...] * inv, wl[:DH], preferred_element_type=jnp.float32)
    z = z + jnp.dot(a1_ref[...] * inv, wl[DH:], preferred_element_type=jnp.float32)
    z = z + jnp.dot(x_ref[...], wr_ref[...], preferred_element_type=jnp.float32)
    h = jnp.maximum(z + b_ref[...], 0.0)
    q0_ref[...] = h[:, 0 * DH:1 * DH]
    q1_ref[...] = h[:, 1 * DH:2 * DH]
    q2_ref[...] = h[:, 2 * DH:3 * DH]
    q3_ref[...] = h[:, 3 * DH:4 * DH]


def _head_body(a0_ref, a1_ref, a2_ref, a3_ref, ca_ref, cb_ref,
               q0_ref, q1_ref, q2_ref, q3_ref,
               wl_ref, wr_ref, b_ref, wlin_ref, blin_ref, o_ref):
    inv = 1.0 / jnp.maximum(ca_ref[:, 0:1] + cb_ref[:, 0:1], 1.0)
    wl = wl_ref[...]
    wr = wr_ref[...]
    z = b_ref[...]
    for i, a in enumerate((a0_ref, a1_ref, a2_ref, a3_ref)):
        z = z + jnp.dot(a[...] * inv, wl[i * DH:(i + 1) * DH],
                        preferred_element_type=jnp.float32)
    for i, q in enumerate((q0_ref, q1_ref, q2_ref, q3_ref)):
        z = z + jnp.dot(q[...], wr[i * DH:(i + 1) * DH],
                        preferred_element_type=jnp.float32)
    h2 = jnp.maximum(z, 0.0)
    o_ref[...] = jnp.dot(h2, wlin_ref[...], preferred_element_type=jnp.float32) + blin_ref[...]


def _row_spec(k):
    return pl.BlockSpec((BLK, k), lambda i: (i, 0))


def _full_spec(r, c):
    return pl.BlockSpec((r, c), lambda i: (0, 0))


def _layer1_tc(a0, a1, ca, cb, x, W_l, W_r, b):
    return pl.pallas_call(
        _layer1_body,
        grid=(N // BLK,),
        in_specs=[_row_spec(DH), _row_spec(DH), _row_spec(DH), _row_spec(DH),
                  _row_spec(2 * DH), _full_spec(2 * DH, 4 * DH),
                  _full_spec(2 * DH, 4 * DH), _full_spec(1, 4 * DH)],
        out_specs=[_row_spec(DH)] * 4,
        out_shape=[jax.ShapeDtypeStruct((N, DH), jnp.float32)] * 4,
    )(a0, a1, ca, cb, x, W_l, W_r, b.reshape(1, -1))


def _head_tc(aggs, ca, cb, qs, W_l, W_r, b, W_lin, b_lin):
    d_out = W_lin.shape[1]
    return pl.pallas_call(
        _head_body,
        grid=(N // BLK,),
        in_specs=[_row_spec(DH)] * 4 + [_row_spec(DH)] * 2 + [_row_spec(DH)] * 4 +
                 [_full_spec(4 * DH, 4 * DH), _full_spec(4 * DH, 4 * DH),
                  _full_spec(1, 4 * DH), _full_spec(4 * DH, d_out),
                  _full_spec(1, d_out)],
        out_specs=_row_spec(d_out),
        out_shape=jax.ShapeDtypeStruct((N, d_out), jnp.float32),
    )(*aggs, ca, cb, *qs, W_l, W_r, b.reshape(1, -1), W_lin, b_lin.reshape(1, -1))


def kernel(x, edge_index, W1_l, b1, W1_r, W2_l, b2, W2_r, W_lin, b_lin):
    src = edge_index[0].astype(jnp.int32)
    dst = edge_index[1].astype(jnp.int32)
    srcp = jnp.pad(src, (0, EP - E))
    pad_dst = N + (jnp.arange(EP - E, dtype=jnp.int32) % 8)
    dstp = jnp.concatenate([dst, pad_dst])

    x0 = x[:, :DH]
    x1 = x[:, DH:]
    zF = jnp.zeros((FCH, DH), jnp.float32)
    oF = jnp.ones((K, DH), jnp.float32)

    agg0, agg1, ca, cb = _agg1_call(x0, x1, srcp, dstp, zF, oF)
    q0, q1, q2, q3 = _layer1_tc(agg0, agg1, ca, cb, x, W1_l, W1_r, b1)

    aggs = _agg2_call(q0, q1, q2, q3, srcp, dstp, zF)
    out = _head_tc(aggs, ca, cb, (q0, q1, q2, q3), W2_l, W2_r, b2, W_lin, b_lin)
    return out


# speedup vs baseline: 3.1757x; 1.0355x over previous
import functools

import jax
import jax.numpy as jnp
from jax import lax
from jax.experimental import pallas as pl
from jax.experimental.pallas import tpu as pltpu
from jax.experimental.pallas import tpu_sc as plsc

N = 10000
E = 160000
EP = 163840
ET = EP // 16
K = 128
NCH = ET // K
NCH0 = 40

DH = 128

ACC = N + 8
FL = 640
FCH = 40

BLK = 400


def _edge_scan(s, src_hbm, dst_hbm, table, acc, srcs, dsts, rows, gsems, ssems):
    def load_and_gather(ci, b):
        base = s * ET + ci * K
        pltpu.sync_copy(src_hbm.at[pl.ds(base, K)], srcs[b])
        pltpu.sync_copy(dst_hbm.at[pl.ds(base, K)], dsts[b])
        pltpu.async_copy(table.at[plsc.Indices(srcs[b])], rows[b], gsems[b])

    def wait_gather(b):
        pltpu.make_async_copy(table.at[plsc.Indices(srcs[b])], rows[b],
                              gsems[b]).wait()

    def start_scatter(b):
        pltpu.async_copy(rows[b], acc.at[plsc.Indices(dsts[b])], ssems[b],
                         add=True)

    def drain_scatter(b):
        pltpu.make_async_copy(rows[b], acc.at[plsc.Indices(dsts[b])],
                              ssems[b]).wait()

    load_and_gather(0, 0)

    def pair(i, carry):
        for b in (0, 1):
            cur = 2 * i + b
            nb = 1 - b
            nxt = cur + 1

            @pl.when(nxt < NCH)
            def _():
                @pl.when(cur >= 1)
                def _():
                    drain_scatter(nb)

                load_and_gather(nxt, nb)

            wait_gather(b)
            start_scatter(b)
        return carry

    lax.fori_loop(0, NCH // 2, pair, 0)
    drain_scatter(0)
    drain_scatter(1)


def _cnt_scan(s, dst_hbm, acc, dsts, ones_v, ssems, ch_lo):
    def start_scatter(b):
        pltpu.async_copy(ones_v, acc.at[plsc.Indices(dsts[b])], ssems[b],
                         add=True)

    def drain_scatter(b):
        pltpu.make_async_copy(ones_v, acc.at[plsc.Indices(dsts[b])],
                              ssems[b]).wait()

    def pair(i, carry):
        for b in (0, 1):
            cur = 2 * i + b

            @pl.when(cur >= 2)
            def _():
                drain_scatter(b)

            base = s * ET + (ch_lo + cur) * K
            pltpu.sync_copy(dst_hbm.at[pl.ds(base, K)], dsts[b])
            start_scatter(b)
        return carry

    lax.fori_loop(0, NCH0 // 2, pair, 0)
    drain_scatter(0)
    drain_scatter(1)


def _zero_acc(start, acc, stage_v):
    for i in range(FL // FCH):
        pltpu.sync_copy(stage_v.at[pl.ds(0, FCH)],
                        acc.at[pl.ds(start + i * FCH, FCH)])


def _flush_acc(start, acc, stage_v, out_hbm):
    for i in range(FL // FCH):
        pltpu.sync_copy(acc.at[pl.ds(start + i * FCH, FCH)],
                        stage_v.at[pl.ds(0, FCH)])
        pltpu.sync_copy(stage_v.at[pl.ds(0, FCH)],
                        out_hbm.at[pl.ds(start + i * FCH, FCH)])


def _agg1_call(x0, x1, srcp, dstp, zF, oF):
    mesh = plsc.VectorSubcoreMesh(core_axis_name="c", subcore_axis_name="s")

    @functools.partial(
        pl.kernel,
        out_type=[jax.ShapeDtypeStruct((N, DH), jnp.float32)] * 4,
        mesh=mesh,
        scratch_types=[
            pltpu.VMEM_SHARED((ACC, DH), jnp.float32),
            pltpu.VMEM((K,), jnp.int32),
            pltpu.VMEM((K,), jnp.int32),
            pltpu.VMEM((K,), jnp.int32),
            pltpu.VMEM((K,), jnp.int32),
            pltpu.VMEM((K, DH), jnp.float32),
            pltpu.VMEM((K, DH), jnp.float32),
            pltpu.VMEM((FCH, DH), jnp.float32),
            pltpu.SemaphoreType.DMA,
            pltpu.SemaphoreType.DMA,
            pltpu.SemaphoreType.DMA,
            pltpu.SemaphoreType.DMA,
        ],
    )
    def k(x0_h, x1_h, src_h, dst_h, zf_h, of_h,
          agg0_o, agg1_o, cnta_o, cntb_o,
          acc, src0_v, src1_v, dst0_v, dst1_v, rows0_v, rows1_v, stage_v,
          gsem0, gsem1, ssem0, ssem1):
        srcs, dsts = (src0_v, src1_v), (dst0_v, dst1_v)
        rows, gsems, ssems = (rows0_v, rows1_v), (gsem0, gsem1), (ssem0, ssem1)
        c = lax.axis_index("c")
        s = lax.axis_index("s")
        start = jnp.minimum(s * FL, N - FL)

        pltpu.sync_copy(zf_h, stage_v)
        _zero_acc(start, acc, stage_v)
        plsc.subcore_barrier()

        @pl.when(c == 0)
        def _():
            _edge_scan(s, src_h, dst_h, x0_h, acc, srcs, dsts, rows, gsems, ssems)

        @pl.when(c == 1)
        def _():
            _edge_scan(s, src_h, dst_h, x1_h, acc, srcs, dsts, rows, gsems, ssems)

        plsc.subcore_barrier()

        @pl.when(c == 0)
        def _():
            _flush_acc(start, acc, stage_v, agg0_o)

        @pl.when(c == 1)
        def _():
            _flush_acc(start, acc, stage_v, agg1_o)

        plsc.subcore_barrier()

        pltpu.sync_copy(zf_h, stage_v)
        _zero_acc(start, acc, stage_v)
        pltpu.sync_copy(of_h, rows0_v)
        plsc.subcore_barrier()

        @pl.when(c == 0)
        def _():
            _cnt_scan(s, dst_h, acc, dsts, rows0_v, ssems, 0)

        @pl.when(c == 1)
        def _():
            _cnt_scan(s, dst_h, acc, dsts, rows0_v, ssems, NCH0)

        plsc.subcore_barrier()

        @pl.when(c == 0)
        def _():
            _flush_acc(start, acc, stage_v, cnta_o)

        @pl.when(c == 1)
        def _():
            _flush_acc(start, acc, stage_v, cntb_o)

    return k(x0, x1, srcp, dstp, zF, oF)


def _agg2_call(q0, q1, q2, q3, srcp, dstp, zF):
    mesh = plsc.VectorSubcoreMesh(core_axis_name="c", subcore_axis_name="s")

    @functools.partial(
        pl.kernel,
        out_type=[jax.ShapeDtypeStruct((N, DH), jnp.float32)] * 4,
        mesh=mesh,
        scratch_types=[
            pltpu.VMEM_SHARED((ACC, DH), jnp.float32),
            pltpu.VMEM((K,), jnp.int32),
            pltpu.VMEM((K,), jnp.int32),
            pltpu.VMEM((K,), jnp.int32),
            pltpu.VMEM((K,), jnp.int32),
            pltpu.VMEM((K, DH), jnp.float32),
            pltpu.VMEM((K, DH), jnp.float32),
            pltpu.VMEM((FCH, DH), jnp.float32),
            pltpu.SemaphoreType.DMA,
            pltpu.SemaphoreType.DMA,
            pltpu.SemaphoreType.DMA,
            pltpu.SemaphoreType.DMA,
        ],
    )
    def k(q0_h, q1_h, q2_h, q3_h, src_h, dst_h, zf_h,
          a0_o, a1_o, a2_o, a3_o,
          acc, src0_v, src1_v, dst0_v, dst1_v, rows0_v, rows1_v, stage_v,
          gsem0, gsem1, ssem0, ssem1):
        srcs, dsts = (src0_v, src1_v), (dst0_v, dst1_v)
        rows, gsems, ssems = (rows0_v, rows1_v), (gsem0, gsem1), (ssem0, ssem1)
        c = lax.axis_index("c")
        s = lax.axis_index("s")
        start = jnp.minimum(s * FL, N - FL)
        for qa, qb, oa, ob in [(q0_h, q2_h, a0_o, a2_o),
                               (q1_h, q3_h, a1_o, a3_o)]:
            pltpu.sync_copy(zf_h, stage_v)
            _zero_acc(start, acc, stage_v)
            plsc.subcore_barrier()

            @pl.when(c == 0)
            def _():
                _edge_scan(s, src_h, dst_h, qa, acc, srcs, dsts, rows, gsems, ssems)

            @pl.when(c == 1)
            def _():
                _edge_scan(s, src_h, dst_h, qb, acc, srcs, dsts, rows, gsems, ssems)

            plsc.subcore_barrier()

            @pl.when(c == 0)
            def _():
                _flush_acc(start, acc, stage_v, oa)

            @pl.when(c == 1)
            def _():
                _flush_acc(start, acc, stage_v, ob)

            plsc.subcore_barrier()

    return k(q0, q1, q2, q3, srcp, dstp, zF)


def _layer1_body(a0_ref, a1_ref, ca_ref, cb_ref, x_ref, wl_ref, wr_ref, b_ref,
                 q0_ref, q1_ref, q2_ref, q3_ref):
    inv = 1.0 / jnp.maximum(ca_ref[:, 0:1] + cb_ref[:, 0:1], 1.0)
    wl = wl_ref[...]
    z = jnp.dot(a0_ref[...] * inv, wl[:DH], preferred_element_type=jnp.float32)
    z = z + jnp.dot(a1_ref[...] * inv, wl[DH:], preferred_element_type=jnp.float32)
    z = z + jnp.dot(x_ref[...], wr_ref[...], preferred_element_type=jnp.float32)
    h = jnp.maximum(z + b_ref[...], 0.0)
    q0_ref[...] = h[:, 0 * DH:1 * DH]
    q1_ref[...] = h[:, 1 * DH:2 * DH]
    q2_ref[...] = h[:, 2 * DH:3 * DH]
    q3_ref[...] = h[:, 3 * DH:4 * DH]


def _head_body(a0_ref, a1_ref, a2_ref, a3_ref, ca_ref, cb_ref,
               q0_ref, q1_ref, q2_ref, q3_ref,
               wl_ref, wr_ref, b_ref, wlin_ref, blin_ref, o_ref):
    inv = 1.0 / jnp.maximum(ca_ref[:, 0:1] + cb_ref[:, 0:1], 1.0)
    wl = wl_ref[...]
    wr = wr_ref[...]
    z = b_ref[...]
    for i, a in enumerate((a0_ref, a1_ref, a2_ref, a3_ref)):
        z = z + jnp.dot(a[...] * inv, wl[i * DH:(i + 1) * DH],
                        preferred_element_type=jnp.float32)
    for i, q in enumerate((q0_ref, q1_ref, q2_ref, q3_ref)):
        z = z + jnp.dot(q[...], wr[i * DH:(i + 1) * DH],
                        preferred_element_type=jnp.float32)
    h2 = jnp.maximum(z, 0.0)
    o_ref[...] = jnp.dot(h2, wlin_ref[...], preferred_element_type=jnp.float32) + blin_ref[...]


def _row_spec(k):
    return pl.BlockSpec((BLK, k), lambda i: (i, 0))


def _full_spec(r, c):
    return pl.BlockSpec((r, c), lambda i: (0, 0))


def _layer1_tc(a0, a1, ca, cb, x, W_l, W_r, b):
    return pl.pallas_call(
        _layer1_body,
        grid=(N // BLK,),
        in_specs=[_row_spec(DH), _row_spec(DH), _row_spec(DH), _row_spec(DH),
                  _row_spec(2 * DH), _full_spec(2 * DH, 4 * DH),
                  _full_spec(2 * DH, 4 * DH), _full_spec(1, 4 * DH)],
        out_specs=[_row_spec(DH)] * 4,
        out_shape=[jax.ShapeDtypeStruct((N, DH), jnp.float32)] * 4,
    )(a0, a1, ca, cb, x, W_l, W_r, b.reshape(1, -1))


def _head_tc(aggs, ca, cb, qs, W_l, W_r, b, W_lin, b_lin):
    d_out = W_lin.shape[1]
    return pl.pallas_call(
        _head_body,
        grid=(N // BLK,),
        in_specs=[_row_spec(DH)] * 4 + [_row_spec(DH)] * 2 + [_row_spec(DH)] * 4 +
                 [_full_spec(4 * DH, 4 * DH), _full_spec(4 * DH, 4 * DH),
                  _full_spec(1, 4 * DH), _full_spec(4 * DH, d_out),
                  _full_spec(1, d_out)],
        out_specs=_row_spec(d_out),
        out_shape=jax.ShapeDtypeStruct((N, d_out), jnp.float32),
    )(*aggs, ca, cb, *qs, W_l, W_r, b.reshape(1, -1), W_lin, b_lin.reshape(1, -1))


def kernel(x, edge_index, W1_l, b1, W1_r, W2_l, b2, W2_r, W_lin, b_lin):
    src = edge_index[0].astype(jnp.int32)
    dst = edge_index[1].astype(jnp.int32)
    srcp = jnp.pad(src, (0, EP - E))
    pad_dst = N + (jnp.arange(EP - E, dtype=jnp.int32) % 8)
    dstp = jnp.concatenate([dst, pad_dst])

    x0 = x[:, :DH]
    x1 = x[:, DH:]
    zF = jnp.zeros((FCH, DH), jnp.float32)
    oF = jnp.ones((K, DH), jnp.float32)

    agg0, agg1, ca, cb = _agg1_call(x0, x1, srcp, dstp, zF, oF)
    q0, q1, q2, q3 = _layer1_tc(agg0, agg1, ca, cb, x, W1_l, W1_r, b1)

    aggs = _agg2_call(q0, q1, q2, q3, srcp, dstp, zF)
    out = _head_tc(aggs, ca, cb, (q0, q1, q2, q3), W2_l, W2_r, b2, W_lin, b_lin)
    return out


# speedup vs baseline: 3.2910x; 1.0363x over previous
import functools

import jax
import jax.numpy as jnp
from jax import lax
from jax.experimental import pallas as pl
from jax.experimental.pallas import tpu as pltpu
from jax.experimental.pallas import tpu_sc as plsc

N = 10000
E = 160000
EP = 163840
ET = EP // 16
K = 128
NCH = ET // K
NCH0 = 40
BPF = 20

DH = 128

ACC = N + 8
FL = 640
FCH = 40

BLK = 400


def _edge_scan(s, src_hbm, dst_hbm, table, acc, bsrc, bdst, dstv, rows,
               gsems, ssems):
    def start_gather(j, b):
        pltpu.async_copy(table.at[plsc.Indices(bsrc.at[pl.ds(j * K, K)])],
                         rows[b], gsems[b])

    def wait_gather(b):
        pltpu.make_async_copy(table.at[plsc.Indices(bsrc.at[pl.ds(0, K)])],
                              rows[b], gsems[b]).wait()

    def copy_dst(j, b):
        for i in range(K // 16):
            dstv[b][pl.ds(i * 16, 16)] = bdst[pl.ds(j * K + i * 16, 16)]

    def start_scatter(b):
        pltpu.async_copy(rows[b], acc.at[plsc.Indices(dstv[b])], ssems[b],
                         add=True)

    def drain_scatter(b):
        pltpu.make_async_copy(rows[b], acc.at[plsc.Indices(dstv[b])],
                              ssems[b]).wait()

    def block(blk, carry):
        base = s * ET + blk * (BPF * K)
        pltpu.sync_copy(src_hbm.at[pl.ds(base, BPF * K)], bsrc)
        pltpu.sync_copy(dst_hbm.at[pl.ds(base, BPF * K)], bdst)
        start_gather(0, 0)

        def pair(i, c2):
            for b in (0, 1):
                j = 2 * i + b
                nb = 1 - b
                nj = j + 1

                @pl.when(nj < BPF)
                def _():
                    @pl.when(j >= 1)
                    def _():
                        drain_scatter(nb)

                    start_gather(nj, nb)

                copy_dst(j, b)
                wait_gather(b)
                start_scatter(b)
            return c2

        lax.fori_loop(0, BPF // 2, pair, 0)
        drain_scatter(0)
        drain_scatter(1)
        return carry

    lax.fori_loop(0, NCH // BPF, block, 0)


def _cnt_scan(s, dst_hbm, acc, bdst, dstv, ones_v, ssems, ch_lo):
    def copy_dst(j, b):
        for i in range(K // 16):
            dstv[b][pl.ds(i * 16, 16)] = bdst[pl.ds(j * K + i * 16, 16)]

    def start_scatter(b):
        pltpu.async_copy(ones_v, acc.at[plsc.Indices(dstv[b])], ssems[b],
                         add=True)

    def drain_scatter(b):
        pltpu.make_async_copy(ones_v, acc.at[plsc.Indices(dstv[b])],
                              ssems[b]).wait()

    def block(blk, carry):
        base = s * ET + (ch_lo + blk * BPF) * K
        pltpu.sync_copy(dst_hbm.at[pl.ds(base, BPF * K)], bdst)

        def pair(i, c2):
            for b in (0, 1):
                j = 2 * i + b

                @pl.when(j >= 2)
                def _():
                    drain_scatter(b)

                copy_dst(j, b)
                start_scatter(b)
            return c2

        lax.fori_loop(0, BPF // 2, pair, 0)
        drain_scatter(0)
        drain_scatter(1)
        return carry

    lax.fori_loop(0, NCH0 // BPF, block, 0)


def _zero_acc(start, acc, stage_v):
    for i in range(FL // FCH):
        pltpu.sync_copy(stage_v.at[pl.ds(0, FCH)],
                        acc.at[pl.ds(start + i * FCH, FCH)])


def _flush_acc(start, acc, stage_v, out_hbm):
    for i in range(FL // FCH):
        pltpu.sync_copy(acc.at[pl.ds(start + i * FCH, FCH)],
                        stage_v.at[pl.ds(0, FCH)])
        pltpu.sync_copy(stage_v.at[pl.ds(0, FCH)],
                        out_hbm.at[pl.ds(start + i * FCH, FCH)])


def _agg1_call(x0, x1, srcp, dstp, zF, oF):
    mesh = plsc.VectorSubcoreMesh(core_axis_name="c", subcore_axis_name="s")

    @functools.partial(
        pl.kernel,
        out_type=[jax.ShapeDtypeStruct((N, DH), jnp.float32)] * 4,
        mesh=mesh,
        scratch_types=[
            pltpu.VMEM_SHARED((ACC, DH), jnp.float32),
            pltpu.VMEM((BPF * K,), jnp.int32),
            pltpu.VMEM((BPF * K,), jnp.int32),
            pltpu.VMEM((K,), jnp.int32),
            pltpu.VMEM((K,), jnp.int32),
            pltpu.VMEM((K, DH), jnp.float32),
            pltpu.VMEM((K, DH), jnp.float32),
            pltpu.VMEM((FCH, DH), jnp.float32),
            pltpu.SemaphoreType.DMA,
            pltpu.SemaphoreType.DMA,
            pltpu.SemaphoreType.DMA,
            pltpu.SemaphoreType.DMA,
        ],
    )
    def k(x0_h, x1_h, src_h, dst_h, zf_h, of_h,
          agg0_o, agg1_o, cnta_o, cntb_o,
          acc, bsrc, bdst, dst0_v, dst1_v, rows0_v, rows1_v, stage_v,
          gsem0, gsem1, ssem0, ssem1):
        dstv = (dst0_v, dst1_v)
        rows, gsems, ssems = (rows0_v, rows1_v), (gsem0, gsem1), (ssem0, ssem1)
        c = lax.axis_index("c")
        s = lax.axis_index("s")
        start = jnp.minimum(s * FL, N - FL)

        pltpu.sync_copy(zf_h, stage_v)
        _zero_acc(start, acc, stage_v)
        plsc.subcore_barrier()

        @pl.when(c == 0)
        def _():
            _edge_scan(s, src_h, dst_h, x0_h, acc, bsrc, bdst, dstv, rows, gsems, ssems)

        @pl.when(c == 1)
        def _():
            _edge_scan(s, src_h, dst_h, x1_h, acc, bsrc, bdst, dstv, rows, gsems, ssems)

        plsc.subcore_barrier()

        @pl.when(c == 0)
        def _():
            _flush_acc(start, acc, stage_v, agg0_o)

        @pl.when(c == 1)
        def _():
            _flush_acc(start, acc, stage_v, agg1_o)

        plsc.subcore_barrier()

        pltpu.sync_copy(zf_h, stage_v)
        _zero_acc(start, acc, stage_v)
        pltpu.sync_copy(of_h, rows0_v)
        plsc.subcore_barrier()

        @pl.when(c == 0)
        def _():
            _cnt_scan(s, dst_h, acc, bdst, dstv, rows0_v, ssems, 0)

        @pl.when(c == 1)
        def _():
            _cnt_scan(s, dst_h, acc, bdst, dstv, rows0_v, ssems, NCH0)

        plsc.subcore_barrier()

        @pl.when(c == 0)
        def _():
            _flush_acc(start, acc, stage_v, cnta_o)

        @pl.when(c == 1)
        def _():
            _flush_acc(start, acc, stage_v, cntb_o)

    return k(x0, x1, srcp, dstp, zF, oF)


def _agg2_call(q0, q1, q2, q3, srcp, dstp, zF):
    mesh = plsc.VectorSubcoreMesh(core_axis_name="c", subcore_axis_name="s")

    @functools.partial(
        pl.kernel,
        out_type=[jax.ShapeDtypeStruct((N, DH), jnp.float32)] * 4,
        mesh=mesh,
        scratch_types=[
            pltpu.VMEM_SHARED((ACC, DH), jnp.float32),
            pltpu.VMEM((BPF * K,), jnp.int32),
            pltpu.VMEM((BPF * K,), jnp.int32),
            pltpu.VMEM((K,), jnp.int32),
            pltpu.VMEM((K,), jnp.int32),
            pltpu.VMEM((K, DH), jnp.float32),
            pltpu.VMEM((K, DH), jnp.float32),
            pltpu.VMEM((FCH, DH), jnp.float32),
            pltpu.SemaphoreType.DMA,
            pltpu.SemaphoreType.DMA,
            pltpu.SemaphoreType.DMA,
            pltpu.SemaphoreType.DMA,
        ],
    )
    def k(q0_h, q1_h, q2_h, q3_h, src_h, dst_h, zf_h,
          a0_o, a1_o, a2_o, a3_o,
          acc, bsrc, bdst, dst0_v, dst1_v, rows0_v, rows1_v, stage_v,
          gsem0, gsem1, ssem0, ssem1):
        dstv = (dst0_v, dst1_v)
        rows, gsems, ssems = (rows0_v, rows1_v), (gsem0, gsem1), (ssem0, ssem1)
        c = lax.axis_index("c")
        s = lax.axis_index("s")
        start = jnp.minimum(s * FL, N - FL)
        for qa, qb, oa, ob in [(q0_h, q2_h, a0_o, a2_o),
                               (q1_h, q3_h, a1_o, a3_o)]:
            pltpu.sync_copy(zf_h, stage_v)
            _zero_acc(start, acc, stage_v)
            plsc.subcore_barrier()

            @pl.when(c == 0)
            def _():
                _edge_scan(s, src_h, dst_h, qa, acc, bsrc, bdst, dstv, rows, gsems, ssems)

            @pl.when(c == 1)
            def _():
                _edge_scan(s, src_h, dst_h, qb, acc, bsrc, bdst, dstv, rows, gsems, ssems)

            plsc.subcore_barrier()

            @pl.when(c == 0)
            def _():
                _flush_acc(start, acc, stage_v, oa)

            @pl.when(c == 1)
            def _():
                _flush_acc(start, acc, stage_v, ob)

            plsc.subcore_barrier()

    return k(q0, q1, q2, q3, srcp, dstp, zF)


def _layer1_body(a0_ref, a1_ref, ca_ref, cb_ref, x_ref, wl_ref, wr_ref, b_ref,
                 q0_ref, q1_ref, q2_ref, q3_ref):
    inv = 1.0 / jnp.maximum(ca_ref[:, 0:1] + cb_ref[:, 0:1], 1.0)
    wl = wl_ref[...]
    z = jnp.dot(a0_ref[...] * inv, wl[:DH], preferred_element_type=jnp.float32)
    z = z + jnp.dot(a1_ref[...] * inv, wl[DH:], preferred_element_type=jnp.float32)
    z = z + jnp.dot(x_ref[...], wr_ref[...], preferred_element_type=jnp.float32)
    h = jnp.maximum(z + b_ref[...], 0.0)
    q0_ref[...] = h[:, 0 * DH:1 * DH]
    q1_ref[...] = h[:, 1 * DH:2 * DH]
    q2_ref[...] = h[:, 2 * DH:3 * DH]
    q3_ref[...] = h[:, 3 * DH:4 * DH]


def _head_body(a0_ref, a1_ref, a2_ref, a3_ref, ca_ref, cb_ref,
               q0_ref, q1_ref, q2_ref, q3_ref,
               wl_ref, wr_ref, b_ref, wlin_ref, blin_ref, o_ref):
    inv = 1.0 / jnp.maximum(ca_ref[:, 0:1] + cb_ref[:, 0:1], 1.0)
    wl = wl_ref[...]
    wr = wr_ref[...]
    z = b_ref[...]
    for i, a in enumerate((a0_ref, a1_ref, a2_ref, a3_ref)):
        z = z + jnp.dot(a[...] * inv, wl[i * DH:(i + 1) * DH],
                        preferred_element_type=jnp.float32)
    for i, q in enumerate((q0_ref, q1_ref, q2_ref, q3_ref)):
        z = z + jnp.dot(q[...], wr[i * DH:(i + 1) * DH],
                        preferred_element_type=jnp.float32)
    h2 = jnp.maximum(z, 0.0)
    o_ref[...] = jnp.dot(h2, wlin_ref[...], preferred_element_type=jnp.float32) + blin_ref[...]


def _row_spec(k):
    return pl.BlockSpec((BLK, k), lambda i: (i, 0))


def _full_spec(r, c):
    return pl.BlockSpec((r, c), lambda i: (0, 0))


def _layer1_tc(a0, a1, ca, cb, x, W_l, W_r, b):
    return pl.pallas_call(
        _layer1_body,
        grid=(N // BLK,),
        in_specs=[_row_spec(DH), _row_spec(DH), _row_spec(DH), _row_spec(DH),
                  _row_spec(2 * DH), _full_spec(2 * DH, 4 * DH),
                  _full_spec(2 * DH, 4 * DH), _full_spec(1, 4 * DH)],
        out_specs=[_row_spec(DH)] * 4,
        out_shape=[jax.ShapeDtypeStruct((N, DH), jnp.float32)] * 4,
    )(a0, a1, ca, cb, x, W_l, W_r, b.reshape(1, -1))


def _head_tc(aggs, ca, cb, qs, W_l, W_r, b, W_lin, b_lin):
    d_out = W_lin.shape[1]
    return pl.pallas_call(
        _head_body,
        grid=(N // BLK,),
        in_specs=[_row_spec(DH)] * 4 + [_row_spec(DH)] * 2 + [_row_spec(DH)] * 4 +
                 [_full_spec(4 * DH, 4 * DH), _full_spec(4 * DH, 4 * DH),
                  _full_spec(1, 4 * DH), _full_spec(4 * DH, d_out),
                  _full_spec(1, d_out)],
        out_specs=_row_spec(d_out),
        out_shape=jax.ShapeDtypeStruct((N, d_out), jnp.float32),
    )(*aggs, ca, cb, *qs, W_l, W_r, b.reshape(1, -1), W_lin, b_lin.reshape(1, -1))


def kernel(x, edge_index, W1_l, b1, W1_r, W2_l, b2, W2_r, W_lin, b_lin):
    src = edge_index[0].astype(jnp.int32)
    dst = edge_index[1].astype(jnp.int32)
    srcp = jnp.pad(src, (0, EP - E))
    pad_dst = N + (jnp.arange(EP - E, dtype=jnp.int32) % 8)
    dstp = jnp.concatenate([dst, pad_dst])

    x0 = x[:, :DH]
    x1 = x[:, DH:]
    zF = jnp.zeros((FCH, DH), jnp.float32)
    oF = jnp.ones((K, DH), jnp.float32)

    agg0, agg1, ca, cb = _agg1_call(x0, x1, srcp, dstp, zF, oF)
    q0, q1, q2, q3 = _layer1_tc(agg0, agg1, ca, cb, x, W1_l, W1_r, b1)

    aggs = _agg2_call(q0, q1, q2, q3, srcp, dstp, zF)
    out = _head_tc(aggs, ca, cb, (q0, q1, q2, q3), W2_l, W2_r, b2, W_lin, b_lin)
    return out
